# baseline (speedup 1.0000x reference)
import jax
import jax.numpy as jnp
from jax import lax
from jax.experimental import pallas as pl
from jax.experimental.pallas import tpu as pltpu
from jax.experimental.pallas import tpu_sc as plsc
import functools

N_NODES = 100000
D_FEAT = 128
NUM_GRAPHS = 1024
OUT_FEATS = 10

NC = 2
NS = 16
NW = NC * NS
ROWS_PER_W = N_NODES // NW
CHUNK = 125
CHUNKS = ROWS_PER_W // CHUNK
CHUNK_PAD = 128
ACC_ROWS = NUM_GRAPHS + 16
ROWS_PER_TILE = NUM_GRAPHS // NS
L = 16


def _sc_segment_sums(x, ids2d):
    mesh = plsc.VectorSubcoreMesh(
        core_axis_name="c", subcore_axis_name="s", num_cores=NC, num_subcores=NS
    )

    @functools.partial(
        pl.kernel,
        out_type=(
            jax.ShapeDtypeStruct((NC, NUM_GRAPHS, D_FEAT), jnp.float32),
            jax.ShapeDtypeStruct((NC, NUM_GRAPHS, L), jnp.float32),
        ),
        mesh=mesh,
        scratch_types=[
            pltpu.VMEM((2, CHUNK_PAD), jnp.int32),
            pltpu.VMEM((2, CHUNK_PAD, D_FEAT), jnp.float32),
            pltpu.VMEM((CHUNK_PAD, D_FEAT), jnp.float32),
            pltpu.VMEM((CHUNK_PAD, L), jnp.float32),
            pltpu.VMEM((CHUNK_PAD,), jnp.int32),
            pltpu.VMEM((CHUNK_PAD // L, L), jnp.int32),
            pltpu.VMEM((ROWS_PER_TILE, D_FEAT), jnp.float32),
            pltpu.VMEM((ROWS_PER_TILE, L), jnp.float32),
            pltpu.VMEM_SHARED((ACC_ROWS, D_FEAT), jnp.float32),
            pltpu.VMEM_SHARED((ACC_ROWS, L), jnp.float32),
            pltpu.SemaphoreType.DMA,
            pltpu.SemaphoreType.DMA,
            pltpu.SemaphoreType.DMA,
            pltpu.SemaphoreType.DMA,
        ],
        compiler_params=pltpu.CompilerParams(
            use_tc_tiling_on_sc=False, needs_layout_passes=False
        ),
    )
    def k(x_hbm, ids_hbm, psums, pcnts, idx_v, rows_v, sum_v, cnt_v, rank_v,
          uniq_v, zsum_v, zcnt_v, acc_sh, cnt_sh, si0, si1, sr0, sr1):
        c = lax.axis_index("c")
        s = lax.axis_index("s")
        wid = c * NS + s
        row0 = wid * ROWS_PER_W
        q0 = wid * CHUNKS
        id_sems = (si0, si1)
        row_sems = (sr0, sr1)

        IOTA = lax.iota(jnp.int32, L)
        ZERO = jnp.zeros((L,), jnp.float32)
        ONE = jnp.ones((L,), jnp.float32)
        FCON = [IOTA + L * j for j in range(D_FEAT // L)]

        def start_gather(kk, slot):
            pltpu.async_copy(ids_hbm.at[q0 + kk], idx_v.at[slot], id_sems[slot])
            pltpu.async_copy(
                x_hbm.at[pl.ds(row0 + kk * CHUNK, CHUNK)],
                rows_v.at[slot, pl.ds(0, CHUNK)],
                row_sems[slot],
            )

        def wait_gather(kk, slot):
            pltpu.make_async_copy(
                ids_hbm.at[q0 + kk], idx_v.at[slot], id_sems[slot]
            ).wait()
            pltpu.make_async_copy(
                x_hbm.at[pl.ds(row0 + kk * CHUNK, CHUNK)],
                rows_v.at[slot, pl.ds(0, CHUNK)],
                row_sems[slot],
            ).wait()

        start_gather(0, 0)
        start_gather(1, 1)

        def init_local(i, carry):
            for j in range(D_FEAT // L):
                sum_v[i, pl.ds(j * L, L)] = ZERO
            cnt_v[i, :] = ZERO
            return carry

        lax.fori_loop(0, CHUNK_PAD, init_local, 0)

        def init_zero(i, carry):
            for j in range(D_FEAT // L):
                zsum_v[i, pl.ds(j * L, L)] = ZERO
            zcnt_v[i, :] = ZERO
            return carry

        lax.fori_loop(0, ROWS_PER_TILE, init_zero, 0)

        for slot in range(2):
            for r in range(CHUNK, CHUNK_PAD):
                for j in range(D_FEAT // L):
                    rows_v[slot, r, pl.ds(j * L, L)] = ZERO

        pltpu.sync_copy(zsum_v, acc_sh.at[pl.ds(s * ROWS_PER_TILE, ROWS_PER_TILE)])
        pltpu.sync_copy(zcnt_v, cnt_sh.at[pl.ds(s * ROWS_PER_TILE, ROWS_PER_TILE)])
        plsc.subcore_barrier()

        def process_chunk(kk, slot):
            wait_gather(kk, slot)

            for kg in range(CHUNK_PAD // L):
                uniq_v[kg, :] = jnp.full((L,), NUM_GRAPHS, jnp.int32)
            slot_full = jnp.full((L,), slot, jnp.int32)
            total = jnp.int32(0)
            for kg in range(CHUNK_PAD // L):
                cur = idx_v[slot, pl.ds(kg * L, L)]
                pidx = jnp.maximum(IOTA + (kg * L - 1), 0)
                prev = plsc.load_gather(idx_v, [slot_full, pidx])
                b = (cur != prev).astype(jnp.int32)
                r = plsc.cumsum(b) + total
                rank_v[pl.ds(kg * L, L)] = r
                plsc.store_scatter(
                    uniq_v,
                    [lax.shift_right_logical(r, 4), lax.bitwise_and(r, 15)],
                    cur,
                )
                total = total + jnp.sum(b)

            def rows_body(g, carry):
                rprev = carry[0]
                rcnt = carry[1]
                accs = list(carry[2:])
                for t in range(4):
                    i = g * 4 + t
                    rspl = plsc.load_gather(
                        rank_v, [jnp.full((L,), 0, jnp.int32) + i]
                    )
                    fresh = rspl != rprev
                    rcnt = jnp.where(fresh, ONE, rcnt + ONE)
                    plsc.store_scatter(cnt_v, [rspl, IOTA], rcnt)
                    for j in range(D_FEAT // L):
                        v = rows_v[slot, i, pl.ds(j * L, L)]
                        accs[j] = jnp.where(fresh, v, accs[j] + v)
                        plsc.store_scatter(sum_v, [rspl, FCON[j]], accs[j])
                    rprev = rspl
                return (rprev, rcnt, *accs)

            minus1 = jnp.full((L,), -1, jnp.int32)
            lax.fori_loop(
                0,
                CHUNK_PAD // 4,
                rows_body,
                (minus1, ZERO, ZERO, ZERO, ZERO, ZERO, ZERO, ZERO, ZERO, ZERO),
            )

            ngroups = lax.shift_right_logical(total + 1 + (L - 1), 4)

            def flush_body(g, carry):
                pltpu.sync_copy(
                    sum_v.at[pl.ds(g * L, L)], acc_sh.at[uniq_v.at[g]], add=True
                )
                pltpu.sync_copy(
                    cnt_v.at[pl.ds(g * L, L)], cnt_sh.at[uniq_v.at[g]], add=True
                )
                return carry

            lax.fori_loop(0, ngroups, flush_body, 0)

        def pair_body(gp, carry):
            for slot in range(2):
                kk = gp * 2 + slot
                process_chunk(kk, slot)

                @pl.when(kk + 2 < CHUNKS)
                def _():
                    start_gather(kk + 2, slot)

            return carry

        lax.fori_loop(0, CHUNKS // 2, pair_body, 0)
        process_chunk(CHUNKS - 1, 0)

        plsc.subcore_barrier()
        sl = pl.ds(s * ROWS_PER_TILE, ROWS_PER_TILE)
        pltpu.sync_copy(acc_sh.at[sl], psums.at[c, sl])
        pltpu.sync_copy(cnt_sh.at[sl], pcnts.at[c, sl])

    return k(x, ids2d)


def _tc_finish(psums, pcnts, W, b2):
    def body(ps, pc, w, b, out):
        sums = ps[0] + ps[1]
        cnt = (pc[0] + pc[1])[:, 0:1]
        mean = sums / jnp.maximum(cnt, 1.0)
        out[...] = (
            lax.dot_general(
                mean, w[...], (((1,), (1,)), ((), ())),
                preferred_element_type=jnp.float32,
            )
            + b[...]
        )

    return pl.pallas_call(
        body,
        out_shape=jax.ShapeDtypeStruct((NUM_GRAPHS, OUT_FEATS), jnp.float32),
    )(psums, pcnts, W, b2)


def kernel(inputs, segment_ids, W, b):
    ids32 = segment_ids.astype(jnp.int32).reshape(NW * CHUNKS, CHUNK)
    ids2d = jnp.pad(
        ids32, ((0, 0), (0, CHUNK_PAD - CHUNK)), constant_values=NUM_GRAPHS
    )
    psums, pcnts = _sc_segment_sums(inputs, ids2d)
    return _tc_finish(psums, pcnts, W, b.reshape(1, OUT_FEATS))


# speedup vs baseline: 2.7834x; 2.7834x over previous
import jax
import jax.numpy as jnp
from jax import lax
from jax.experimental import pallas as pl
from jax.experimental.pallas import tpu as pltpu
from jax.experimental.pallas import tpu_sc as plsc
import functools

N_NODES = 100000
D_FEAT = 128
NUM_GRAPHS = 1024
OUT_FEATS = 10

NC = 2
NS = 16
NW = NC * NS
ROWS_PER_W = N_NODES // NW
CHUNK = 125
CHUNKS = ROWS_PER_W // CHUNK
CHUNK_PAD = 128
ACC_ROWS = NUM_GRAPHS + 16
ROWS_PER_TILE = NUM_GRAPHS // NS


def _sc_segment_sums(x, ids2d):
    mesh = plsc.VectorSubcoreMesh(
        core_axis_name="c", subcore_axis_name="s", num_cores=NC, num_subcores=NS
    )

    @functools.partial(
        pl.kernel,
        out_type=(
            jax.ShapeDtypeStruct((NC, NUM_GRAPHS, D_FEAT), jnp.float32),
            jax.ShapeDtypeStruct((NC, NUM_GRAPHS, 16), jnp.float32),
        ),
        mesh=mesh,
        scratch_types=[
            pltpu.VMEM((2, CHUNK_PAD), jnp.int32),
            pltpu.VMEM((2, CHUNK_PAD, D_FEAT), jnp.float32),
            pltpu.VMEM((CHUNK_PAD, 16), jnp.float32),
            pltpu.VMEM((ROWS_PER_TILE, D_FEAT), jnp.float32),
            pltpu.VMEM((ROWS_PER_TILE, 16), jnp.float32),
            pltpu.VMEM_SHARED((ACC_ROWS, D_FEAT), jnp.float32),
            pltpu.VMEM_SHARED((ACC_ROWS, 16), jnp.float32),
            pltpu.SemaphoreType.DMA,
            pltpu.SemaphoreType.DMA,
            pltpu.SemaphoreType.DMA,
            pltpu.SemaphoreType.DMA,
        ],
        compiler_params=pltpu.CompilerParams(use_tc_tiling_on_sc=False),
    )
    def k(x_hbm, ids_hbm, psums, pcnts, idx_v, rows_v, ones_v, zsum_v, zcnt_v,
          acc_sh, cnt_sh, si0, si1, sr0, sr1):
        c = lax.axis_index("c")
        s = lax.axis_index("s")
        wid = c * NS + s
        row0 = wid * ROWS_PER_W
        q0 = wid * CHUNKS
        id_sems = (si0, si1)
        row_sems = (sr0, sr1)

        def start_gather(kk, slot):
            pltpu.async_copy(
                ids_hbm.at[q0 + kk], idx_v.at[slot], id_sems[slot]
            )
            pltpu.async_copy(
                x_hbm.at[pl.ds(row0 + kk * CHUNK, CHUNK)],
                rows_v.at[slot, pl.ds(0, CHUNK)],
                row_sems[slot],
            )

        def wait_gather(kk, slot):
            pltpu.make_async_copy(
                ids_hbm.at[q0 + kk], idx_v.at[slot], id_sems[slot]
            ).wait()
            pltpu.make_async_copy(
                x_hbm.at[pl.ds(row0 + kk * CHUNK, CHUNK)],
                rows_v.at[slot, pl.ds(0, CHUNK)],
                row_sems[slot],
            ).wait()

        start_gather(0, 0)
        start_gather(1, 1)

        def init_ones(i, carry):
            ones_v[i, :] = jnp.ones((16,), jnp.float32)
            return carry

        lax.fori_loop(0, CHUNK_PAD, init_ones, 0)

        def init_zero(i, carry):
            for j in range(D_FEAT // 16):
                zsum_v[i, pl.ds(j * 16, 16)] = jnp.zeros((16,), jnp.float32)
            zcnt_v[i, :] = jnp.zeros((16,), jnp.float32)
            return carry

        lax.fori_loop(0, ROWS_PER_TILE, init_zero, 0)

        for slot in range(2):
            for r in range(CHUNK, CHUNK_PAD):
                for j in range(D_FEAT // 16):
                    rows_v[slot, r, pl.ds(j * 16, 16)] = jnp.zeros(
                        (16,), jnp.float32
                    )

        pltpu.sync_copy(zsum_v, acc_sh.at[pl.ds(s * ROWS_PER_TILE, ROWS_PER_TILE)])
        pltpu.sync_copy(zcnt_v, cnt_sh.at[pl.ds(s * ROWS_PER_TILE, ROWS_PER_TILE)])
        plsc.subcore_barrier()

        for kk in range(CHUNKS):
            slot = kk % 2
            wait_gather(kk, slot)
            pltpu.sync_copy(rows_v.at[slot], acc_sh.at[idx_v.at[slot]], add=True)
            pltpu.sync_copy(ones_v, cnt_sh.at[idx_v.at[slot]], add=True)
            if kk + 2 < CHUNKS:
                start_gather(kk + 2, slot)
        plsc.subcore_barrier()

        sl = pl.ds(s * ROWS_PER_TILE, ROWS_PER_TILE)
        pltpu.sync_copy(acc_sh.at[sl], psums.at[c, sl])
        pltpu.sync_copy(cnt_sh.at[sl], pcnts.at[c, sl])

    return k(x, ids2d)


def _tc_finish(psums, pcnts, W, b2):
    def body(ps_hbm, pc_hbm, w, b, out, ps, pc, sem1, sem2):
        cp1 = pltpu.make_async_copy(ps_hbm, ps, sem1)
        cp2 = pltpu.make_async_copy(pc_hbm, pc, sem2)
        cp1.start()
        cp2.start()
        cp1.wait()
        cp2.wait()
        sums = ps[0] + ps[1]
        cnt = (pc[0] + pc[1])[:, 0:1]
        mean = sums / jnp.maximum(cnt, 1.0)
        out[...] = (
            lax.dot_general(
                mean, w[...], (((1,), (1,)), ((), ())),
                preferred_element_type=jnp.float32,
            )
            + b[...]
        )

    return pl.pallas_call(
        body,
        in_specs=[
            pl.BlockSpec(memory_space=pltpu.HBM),
            pl.BlockSpec(memory_space=pltpu.HBM),
            pl.BlockSpec(memory_space=pltpu.VMEM),
            pl.BlockSpec(memory_space=pltpu.VMEM),
        ],
        scratch_shapes=[
            pltpu.VMEM((NC, NUM_GRAPHS, D_FEAT), jnp.float32),
            pltpu.VMEM((NC, NUM_GRAPHS, 16), jnp.float32),
            pltpu.SemaphoreType.DMA,
            pltpu.SemaphoreType.DMA,
        ],
        out_shape=jax.ShapeDtypeStruct((NUM_GRAPHS, OUT_FEATS), jnp.float32),
    )(psums, pcnts, W, b2)


def kernel(inputs, segment_ids, W, b):
    ids32 = segment_ids.astype(jnp.int32).reshape(NW * CHUNKS, CHUNK)
    ids2d = jnp.pad(
        ids32, ((0, 0), (0, CHUNK_PAD - CHUNK)), constant_values=NUM_GRAPHS
    )
    psums, pcnts = _sc_segment_sums(inputs, ids2d)
    return _tc_finish(psums, pcnts, W, b.reshape(1, OUT_FEATS))
